# baseline (speedup 1.0000x reference)
import jax
import jax.numpy as jnp
from jax import lax
from jax.experimental import pallas as pl
from jax.experimental.pallas import tpu as pltpu
from jax.experimental.pallas import tpu_sc as plsc

B = 16384
CAT = 26
DN = 13
VOC = 100001
DT = 32
NTOK = DN + CAT

NC = 2
NS = 16
NW = NC * NS
BPW = B // NW
C = 32
NCHUNK = BPW // C
GROUP = 4
NG = (CAT + GROUP - 1) // GROUP


def _tok_body(xcat_hbm, xnum_hbm, w_hbm, bias_hbm, *rest):
    tabs = rest[:NG]
    (out_hbm, xidx_v, rows_v, all_v, xnum_v, w_v, bias_v, sem) = rest[NG:]
    wid = lax.axis_index("s") * NC + lax.axis_index("c")

    pltpu.sync_copy(w_hbm, w_v)
    pltpu.sync_copy(bias_hbm, bias_v)

    def chunk_body(ci, carry):
        gb = (wid * NCHUNK + ci) * C

        pltpu.sync_copy(xcat_hbm.at[:, pl.ds(gb, C)], xidx_v)

        for f in range(CAT):
            fo = f % GROUP
            if fo:
                for k in range(C // 16):
                    s = pl.ds(k * 16, 16)
                    xidx_v[f, s] = xidx_v[f, s] + jnp.int32(fo * VOC)

        cps = []
        for f in range(CAT):
            cp = pltpu.make_async_copy(
                tabs[f // GROUP].at[xidx_v.at[f]],
                rows_v.at[pl.ds(f * C, C)], sem)
            cp.start()
            cps.append(cp)

        pltpu.sync_copy(xnum_hbm.at[pl.ds(gb * 16, C * 16)], xnum_v)

        for cp in cps:
            cp.wait()

        def b_body(b, carry2):
            base = b * (NTOK * DT)
            xrow = xnum_v[pl.ds(b * 16, 16)]
            for d in range(DN):
                xi = xrow[d]
                for h in range(DT // 16):
                    s = pl.ds(base + d * DT + h * 16, 16)
                    sw = pl.ds(d * DT + h * 16, 16)
                    all_v[s] = xi * w_v[sw] + bias_v[sw]
            for f in range(CAT):
                for h in range(DT // 16):
                    s = pl.ds(base + (DN + f) * DT + h * 16, 16)
                    sb = pl.ds((DN + f) * DT + h * 16, 16)
                    all_v[s] = rows_v[f * C + b, pl.ds(h * 16, 16)] + bias_v[sb]
            return carry2

        lax.fori_loop(0, C, b_body, 0)

        pltpu.sync_copy(all_v, out_hbm.at[pl.ds(gb * (NTOK * DT), C * NTOK * DT)])
        return carry

    lax.fori_loop(0, NCHUNK, chunk_body, 0)


def kernel(x_cat, x_num, weight, bias, tables):
    xcat_t = x_cat.astype(jnp.int32).T
    x_num16 = jnp.pad(x_num, ((0, 0), (0, 16 - DN))).reshape(B * 16)
    tabs = [tables[g * GROUP:(g + 1) * GROUP].reshape(-1, DT) for g in range(NG)]

    tok = pl.kernel(
        _tok_body,
        out_type=jax.ShapeDtypeStruct((B * NTOK * DT,), jnp.float32),
        mesh=plsc.VectorSubcoreMesh(core_axis_name="c", subcore_axis_name="s"),
        compiler_params=pltpu.CompilerParams(use_tc_tiling_on_sc=False),
        scratch_types=[
            pltpu.VMEM((CAT, C), jnp.int32),
            pltpu.VMEM((CAT * C, DT), jnp.float32),
            pltpu.VMEM((C * NTOK * DT,), jnp.float32),
            pltpu.VMEM((C * 16,), jnp.float32),
            pltpu.VMEM((DN * DT,), jnp.float32),
            pltpu.VMEM((NTOK * DT,), jnp.float32),
            pltpu.SemaphoreType.DMA,
        ],
    )
    out = tok(xcat_t, x_num16, weight.reshape(DN * DT), bias.reshape(NTOK * DT),
              *tabs)
    return out.reshape(B, NTOK, DT)


# speedup vs baseline: 1.0368x; 1.0368x over previous
import jax
import jax.numpy as jnp
from jax import lax
from jax.experimental import pallas as pl
from jax.experimental.pallas import tpu as pltpu
from jax.experimental.pallas import tpu_sc as plsc

B = 16384
CAT = 26
DN = 13
VOC = 100001
DT = 32
NTOK = DN + CAT

NC = 2
NS = 16
NW = NC * NS
BPW = B // NW
C = 32
NCHUNK = BPW // C
GROUP = 1
NG = (CAT + GROUP - 1) // GROUP


def _tok_body(xcat_hbm, xnum_hbm, w_hbm, bias_hbm, *rest):
    tabs = rest[:NG]
    (out_hbm, xidx_v, rows_v, all_v, xnum_v, w_v, bias_v, sem) = rest[NG:]
    wid = lax.axis_index("s") * NC + lax.axis_index("c")

    pltpu.sync_copy(w_hbm, w_v)
    pltpu.sync_copy(bias_hbm, bias_v)

    def chunk_body(ci, carry):
        gb = (wid * NCHUNK + ci) * C

        pltpu.sync_copy(xcat_hbm.at[:, pl.ds(gb, C)], xidx_v)

        for f in range(CAT):
            fo = f % GROUP
            if fo:
                for k in range(C // 16):
                    s = pl.ds(k * 16, 16)
                    xidx_v[f, s] = xidx_v[f, s] + jnp.int32(fo * VOC)

        cps = []
        for f in range(CAT):
            cp = pltpu.make_async_copy(
                tabs[f // GROUP].at[xidx_v.at[f]],
                rows_v.at[pl.ds(f * C, C)], sem)
            cp.start()
            cps.append(cp)

        pltpu.sync_copy(xnum_hbm.at[pl.ds(gb * 16, C * 16)], xnum_v)

        for cp in cps:
            cp.wait()

        def b_body(b, carry2):
            base = b * (NTOK * DT)
            xrow = xnum_v[pl.ds(b * 16, 16)]
            for d in range(DN):
                xi = xrow[d]
                for h in range(DT // 16):
                    s = pl.ds(base + d * DT + h * 16, 16)
                    sw = pl.ds(d * DT + h * 16, 16)
                    all_v[s] = xi * w_v[sw] + bias_v[sw]
            for f in range(CAT):
                for h in range(DT // 16):
                    s = pl.ds(base + (DN + f) * DT + h * 16, 16)
                    sb = pl.ds((DN + f) * DT + h * 16, 16)
                    all_v[s] = rows_v[f * C + b, pl.ds(h * 16, 16)] + bias_v[sb]
            return carry2

        lax.fori_loop(0, C, b_body, 0)

        pltpu.sync_copy(all_v, out_hbm.at[pl.ds(gb * (NTOK * DT), C * NTOK * DT)])
        return carry

    lax.fori_loop(0, NCHUNK, chunk_body, 0)


def kernel(x_cat, x_num, weight, bias, tables):
    xcat_t = x_cat.astype(jnp.int32).T
    x_num16 = jnp.pad(x_num, ((0, 0), (0, 16 - DN))).reshape(B * 16)
    tabs = [tables[g * GROUP:(g + 1) * GROUP].reshape(-1, DT) for g in range(NG)]

    tok = pl.kernel(
        _tok_body,
        out_type=jax.ShapeDtypeStruct((B * NTOK * DT,), jnp.float32),
        mesh=plsc.VectorSubcoreMesh(core_axis_name="c", subcore_axis_name="s"),
        compiler_params=pltpu.CompilerParams(use_tc_tiling_on_sc=False),
        scratch_types=[
            pltpu.VMEM((CAT, C), jnp.int32),
            pltpu.VMEM((CAT * C, DT), jnp.float32),
            pltpu.VMEM((C * NTOK * DT,), jnp.float32),
            pltpu.VMEM((C * 16,), jnp.float32),
            pltpu.VMEM((DN * DT,), jnp.float32),
            pltpu.VMEM((NTOK * DT,), jnp.float32),
            pltpu.SemaphoreType.DMA,
        ],
    )
    out = tok(xcat_t, x_num16, weight.reshape(DN * DT), bias.reshape(NTOK * DT),
              *tabs)
    return out.reshape(B, NTOK, DT)
